# baseline (speedup 1.0000x reference)
import jax
import jax.numpy as jnp
from jax.experimental import pallas as pl
from jax.experimental.pallas import tpu as pltpu

B, N, NFEAT, NHID = 2, 4096, 128, 128
TILE = 256
T = N // TILE


def _layer1_body(x_ref, adj_ref, w1_ref, b1_ref, w2_ref, s2_ref, gh1_ref,
                 s1_ref):
    t = pl.program_id(1)

    @pl.when(t == 0)
    def _init():
        s1_ref[...] = jnp.dot(x_ref[0], w1_ref[...],
                              preferred_element_type=jnp.float32)
        gh1_ref[...] = jnp.zeros_like(gh1_ref)

    h1 = jnp.maximum(
        jnp.dot(adj_ref[0], s1_ref[...],
                preferred_element_type=jnp.float32) + b1_ref[...], 0.0)
    s2_ref[0] = jnp.dot(h1, w2_ref[...], preferred_element_type=jnp.float32)
    gh1_ref[0] += jnp.sum(h1, axis=0, keepdims=True)


def _layer2_body(adj_ref, s2_ref, b2_ref, h2_ref, gh2_ref):
    t = pl.program_id(1)

    @pl.when(t == 0)
    def _init():
        gh2_ref[...] = jnp.zeros_like(gh2_ref)

    h2 = jnp.maximum(
        jnp.dot(adj_ref[0], s2_ref[0],
                preferred_element_type=jnp.float32) + b2_ref[...], 0.0)
    h2_ref[0] = h2
    gh2_ref[0] += jnp.sum(h2, axis=0, keepdims=True)


def kernel(x, adj, W1, b1, W2, b2):
    b1r = b1.reshape(1, NHID)
    b2r = b2.reshape(1, NHID)

    s2, gh1 = pl.pallas_call(
        _layer1_body,
        grid=(B, T),
        in_specs=[
            pl.BlockSpec((1, N, NFEAT), lambda b, t: (b, 0, 0)),
            pl.BlockSpec((1, TILE, N), lambda b, t: (b, t, 0)),
            pl.BlockSpec((NFEAT, NHID), lambda b, t: (0, 0)),
            pl.BlockSpec((1, NHID), lambda b, t: (0, 0)),
            pl.BlockSpec((NHID, NHID), lambda b, t: (0, 0)),
        ],
        out_specs=[
            pl.BlockSpec((1, TILE, NHID), lambda b, t: (b, t, 0)),
            pl.BlockSpec((1, 1, NHID), lambda b, t: (b, 0, 0)),
        ],
        out_shape=[
            jax.ShapeDtypeStruct((B, N, NHID), jnp.float32),
            jax.ShapeDtypeStruct((B, 1, NHID), jnp.float32),
        ],
        scratch_shapes=[pltpu.VMEM((N, NHID), jnp.float32)],
    )(x, adj, W1, b1r, W2)

    h2, gh2 = pl.pallas_call(
        _layer2_body,
        grid=(B, T),
        in_specs=[
            pl.BlockSpec((1, TILE, N), lambda b, t: (b, t, 0)),
            pl.BlockSpec((1, N, NHID), lambda b, t: (b, 0, 0)),
            pl.BlockSpec((1, NHID), lambda b, t: (0, 0)),
        ],
        out_specs=[
            pl.BlockSpec((1, TILE, NHID), lambda b, t: (b, t, 0)),
            pl.BlockSpec((1, 1, NHID), lambda b, t: (b, 0, 0)),
        ],
        out_shape=[
            jax.ShapeDtypeStruct((B, N, NHID), jnp.float32),
            jax.ShapeDtypeStruct((B, 1, NHID), jnp.float32),
        ],
    )(adj, s2, b2r)

    gh = jnp.concatenate([gh1[:, 0, :], gh2[:, 0, :]], axis=-1)
    return (h2, gh)


# speedup vs baseline: 1.2187x; 1.2187x over previous
import jax
import jax.numpy as jnp
from jax.experimental import pallas as pl
from jax.experimental.pallas import tpu as pltpu

B, N, NFEAT, NHID = 2, 4096, 128, 128
TILE = 512
T = N // TILE


def _layer1_body(x_ref, adj_ref, w1_ref, b1_ref, w2_ref, s2_ref, gh1_ref,
                 s1_ref):
    t = pl.program_id(1)

    @pl.when(t == 0)
    def _init():
        s1_ref[...] = jnp.dot(x_ref[0], w1_ref[...],
                              preferred_element_type=jnp.float32
                              ).astype(jnp.bfloat16)
        gh1_ref[...] = jnp.zeros_like(gh1_ref)

    h1 = jnp.maximum(
        jnp.dot(adj_ref[0].astype(jnp.bfloat16), s1_ref[...],
                preferred_element_type=jnp.float32) + b1_ref[...], 0.0)
    s2_ref[0] = jnp.dot(h1, w2_ref[...],
                        preferred_element_type=jnp.float32
                        ).astype(jnp.bfloat16)
    gh1_ref[0] += jnp.sum(h1, axis=0, keepdims=True)


def _layer2_body(adj_ref, s2_ref, b2_ref, h2_ref, gh2_ref):
    t = pl.program_id(1)

    @pl.when(t == 0)
    def _init():
        gh2_ref[...] = jnp.zeros_like(gh2_ref)

    h2 = jnp.maximum(
        jnp.dot(adj_ref[0].astype(jnp.bfloat16), s2_ref[0],
                preferred_element_type=jnp.float32) + b2_ref[...], 0.0)
    h2_ref[0] = h2
    gh2_ref[0] += jnp.sum(h2, axis=0, keepdims=True)


def kernel(x, adj, W1, b1, W2, b2):
    b1r = b1.reshape(1, NHID)
    b2r = b2.reshape(1, NHID)

    s2, gh1 = pl.pallas_call(
        _layer1_body,
        grid=(B, T),
        in_specs=[
            pl.BlockSpec((1, N, NFEAT), lambda b, t: (b, 0, 0)),
            pl.BlockSpec((1, TILE, N), lambda b, t: (b, t, 0)),
            pl.BlockSpec((NFEAT, NHID), lambda b, t: (0, 0)),
            pl.BlockSpec((1, NHID), lambda b, t: (0, 0)),
            pl.BlockSpec((NHID, NHID), lambda b, t: (0, 0)),
        ],
        out_specs=[
            pl.BlockSpec((1, TILE, NHID), lambda b, t: (b, t, 0)),
            pl.BlockSpec((1, 1, NHID), lambda b, t: (b, 0, 0)),
        ],
        out_shape=[
            jax.ShapeDtypeStruct((B, N, NHID), jnp.bfloat16),
            jax.ShapeDtypeStruct((B, 1, NHID), jnp.float32),
        ],
        scratch_shapes=[pltpu.VMEM((N, NHID), jnp.bfloat16)],
    )(x, adj, W1, b1r, W2)

    h2, gh2 = pl.pallas_call(
        _layer2_body,
        grid=(B, T),
        in_specs=[
            pl.BlockSpec((1, TILE, N), lambda b, t: (b, t, 0)),
            pl.BlockSpec((1, N, NHID), lambda b, t: (b, 0, 0)),
            pl.BlockSpec((1, NHID), lambda b, t: (0, 0)),
        ],
        out_specs=[
            pl.BlockSpec((1, TILE, NHID), lambda b, t: (b, t, 0)),
            pl.BlockSpec((1, 1, NHID), lambda b, t: (b, 0, 0)),
        ],
        out_shape=[
            jax.ShapeDtypeStruct((B, N, NHID), jnp.float32),
            jax.ShapeDtypeStruct((B, 1, NHID), jnp.float32),
        ],
    )(adj, s2, b2r)

    gh = jnp.concatenate([gh1[:, 0, :], gh2[:, 0, :]], axis=-1)
    return (h2, gh)
